# Initial kernel scaffold; baseline (speedup 1.0000x reference)
import jax
import jax.numpy as jnp
from jax.experimental import pallas as pl


def kernel(x, edge_index, edge_attr, Wf, bf, Wq, Wk, Wa, Wu, bu):
    raise NotImplementedError("write your pallas kernel here")


# speedup vs baseline: 1.6651x; 1.6651x over previous
import functools

import jax
import jax.numpy as jnp
from jax import lax
from jax.experimental import pallas as pl
from jax.experimental.pallas import tpu as pltpu
from jax.experimental.pallas import tpu_sc as plsc

N = 10000
E = 320000
D = 128
EF = 65
EFP = 128

NC = 2
NS = 16
NW = NC * NS
EPT = E // NW
EB = 80
NBLK = EPT // EB
NP = 10240

QK = float(D) ** (-0.5)


def _node_prep_body(x_ref, w1_ref, w2_ref, w3_ref, wq_ref,
                    t1_ref, t2_ref, tq_ref):
    xb = x_ref[...]
    a1 = jnp.dot(xb, w1_ref[...], preferred_element_type=jnp.float32)
    a2 = jnp.dot(xb, w2_ref[...], preferred_element_type=jnp.float32)
    a3 = jnp.dot(xb, w3_ref[...], preferred_element_type=jnp.float32)
    t1_ref[...] = a1 + a3
    t2_ref[...] = a2 - a3
    tq_ref[...] = jnp.dot(xb, wq_ref[...], preferred_element_type=jnp.float32)


def _node_prep(x, w1, w2, w3, wq):
    nb = 400
    grid = N // nb
    full = lambda shape: pl.BlockSpec(shape, lambda i: (0,) * len(shape))
    return pl.pallas_call(
        _node_prep_body,
        grid=(grid,),
        in_specs=[
            pl.BlockSpec((nb, D), lambda i: (i, 0)),
            full((D, D)), full((D, D)), full((D, D)), full((D, D)),
        ],
        out_specs=[
            pl.BlockSpec((nb, D), lambda i: (i, 0)),
            pl.BlockSpec((nb, D), lambda i: (i, 0)),
            pl.BlockSpec((nb, D), lambda i: (i, 0)),
        ],
        out_shape=[
            jax.ShapeDtypeStruct((N, D), jnp.float32),
            jax.ShapeDtypeStruct((N, D), jnp.float32),
            jax.ShapeDtypeStruct((N, D), jnp.float32),
        ],
    )(x, w1, w2, w3, wq)


def _edge_prep_body(ea_ref, w4_ref, bf_ref, c_ref, m_ref):
    eb = ea_ref[...]
    c_ref[...] = jnp.dot(eb, w4_ref[...],
                         preferred_element_type=jnp.float32) + bf_ref[...]
    m_ref[...] = (eb[:, 0] < 8.0).astype(jnp.float32)


def _edge_prep(ea_pad, w4_pad, bf):
    ebk = 512
    grid = E // ebk
    return pl.pallas_call(
        _edge_prep_body,
        grid=(grid,),
        in_specs=[
            pl.BlockSpec((ebk, EFP), lambda i: (i, 0)),
            pl.BlockSpec((EFP, D), lambda i: (0, 0)),
            pl.BlockSpec((1, D), lambda i: (0, 0)),
        ],
        out_specs=[
            pl.BlockSpec((ebk, D), lambda i: (i, 0)),
            pl.BlockSpec((ebk,), lambda i: (i,)),
        ],
        out_shape=[
            jax.ShapeDtypeStruct((E, D), jnp.float32),
            jax.ShapeDtypeStruct((E,), jnp.float32),
        ],
    )(ea_pad, w4_pad, bf)


def _kmm_body(f_ref, wk_ref, k_ref):
    k_ref[...] = jnp.dot(f_ref[...], wk_ref[...],
                         preferred_element_type=jnp.float32)


def _kmm(f, wk):
    ebk = 512
    grid = E // ebk
    return pl.pallas_call(
        _kmm_body,
        grid=(grid,),
        in_specs=[
            pl.BlockSpec((ebk, D), lambda i: (i, 0)),
            pl.BlockSpec((D, D), lambda i: (0, 0)),
        ],
        out_specs=pl.BlockSpec((ebk, D), lambda i: (i, 0)),
        out_shape=jax.ShapeDtypeStruct((E, D), jnp.float32),
    )(f, wk)


def _final_body(x_ref, ag_ref, wu_ref, bu_ref, o_ref):
    xa = x_ref[...] + ag_ref[0] + ag_ref[1]
    o_ref[...] = jnp.dot(xa, wu_ref[...],
                         preferred_element_type=jnp.float32) + bu_ref[...]


def _final(x, aggr, wu, bu):
    nb = 400
    grid = N // nb
    return pl.pallas_call(
        _final_body,
        grid=(grid,),
        in_specs=[
            pl.BlockSpec((nb, D), lambda i: (i, 0)),
            pl.BlockSpec((2, nb, D), lambda i: (0, i, 0)),
            pl.BlockSpec((D, D), lambda i: (0, 0)),
            pl.BlockSpec((1, D), lambda i: (0, 0)),
        ],
        out_specs=pl.BlockSpec((nb, D), lambda i: (i, 0)),
        out_shape=jax.ShapeDtypeStruct((N, D), jnp.float32),
    )(x, aggr, wu, bu)


def _sc1_body(t1_hbm, t2_hbm, c_hbm, mask_hbm, src_hbm, dst_hbm,
              f_out,
              srcv, dstv, s_v, d_v, c_v, mask_v, f_v):
    cid = lax.axis_index("c")
    sid = lax.axis_index("s")
    tile_base = (cid * NS + sid) * EPT

    def block(blk, carry):
        e0 = tile_base + blk * EB
        pltpu.sync_copy(src_hbm.at[pl.ds(e0, EB)], srcv.at[0])
        pltpu.sync_copy(dst_hbm.at[pl.ds(e0, EB)], dstv.at[0])
        pltpu.sync_copy(t1_hbm.at[srcv.at[0]], s_v)
        pltpu.sync_copy(t2_hbm.at[dstv.at[0]], d_v)
        pltpu.sync_copy(c_hbm.at[pl.ds(e0, EB)], c_v)
        pltpu.sync_copy(mask_hbm.at[pl.ds(e0, EB)], mask_v)

        def group(g, c2):
            gbase = g * 16
            m16 = mask_v[pl.ds(gbase, 16)]
            for j in range(16):
                e = gbase + j
                me = m16[j]
                for c in range(8):
                    sl = pl.ds(c * 16, 16)
                    pre = s_v[e, sl] + d_v[e, sl] + c_v[e, sl]
                    f_v[e, sl] = jnp.where(pre > 0.0, pre,
                                           jnp.exp(pre) - 1.0) * me
            return c2

        lax.fori_loop(0, EB // 16, group, 0)
        pltpu.sync_copy(f_v, f_out.at[pl.ds(e0, EB)])
        return carry

    lax.fori_loop(0, NBLK, block, 0)


def _sc1(t1, t2, c, mask, src, dst):
    mesh = plsc.VectorSubcoreMesh(core_axis_name="c", subcore_axis_name="s")
    fn = functools.partial(
        pl.kernel,
        out_type=jax.ShapeDtypeStruct((E, D), jnp.float32),
        mesh=mesh,
        compiler_params=pltpu.CompilerParams(needs_layout_passes=False),
        scratch_types=[
            pltpu.VMEM((1, EB), jnp.int32),
            pltpu.VMEM((1, EB), jnp.int32),
            pltpu.VMEM((EB, D), jnp.float32),
            pltpu.VMEM((EB, D), jnp.float32),
            pltpu.VMEM((EB, D), jnp.float32),
            pltpu.VMEM((EB,), jnp.float32),
            pltpu.VMEM((EB, D), jnp.float32),
        ],
    )(_sc1_body)
    return fn(t1, t2, c, mask, src, dst)


def _bf16rne(v):
    u = plsc.bitcast(v, jnp.uint32)
    lsb = jnp.right_shift(u, jnp.uint32(16)) & jnp.uint32(1)
    r = (u + jnp.uint32(0x7FFF) + lsb) & jnp.uint32(0xFFFF0000)
    return plsc.bitcast(r, jnp.float32)


def _sc2_body(tq_hbm, k_hbm, src_hbm, wab_hbm,
              a_out, asum_out,
              srcv, q_v, k_v, a_v, wab_v, tmp_v, tbuf_v, asum_sp):
    cid = lax.axis_index("c")
    sid = lax.axis_index("s")
    tile_base = (cid * NS + sid) * EPT

    pltpu.sync_copy(wab_hbm, wab_v)
    for i in range(40):
        tmp_v[pl.ds(i * 16, 16)] = jnp.zeros((16,), jnp.float32)
    pltpu.sync_copy(tmp_v, asum_sp.at[pl.ds(sid * 624, 640)])
    plsc.subcore_barrier()

    def block(blk, carry):
        e0 = tile_base + blk * EB
        pltpu.sync_copy(src_hbm.at[pl.ds(e0, EB)], srcv.at[0])
        pltpu.sync_copy(tq_hbm.at[srcv.at[0]], q_v)
        pltpu.sync_copy(k_hbm.at[pl.ds(e0, EB)], k_v)

        lanes = lax.iota(jnp.int32, 16)

        def group(g, c2):
            gbase = g * 16
            for j in range(16):
                e = gbase + j
                acc = jnp.zeros((16,), jnp.float32)
                for c in range(8):
                    sl = pl.ds(c * 16, 16)
                    v = (q_v[e, sl] * k_v[e, sl]) * jnp.float32(QK)
                    acc = acc + _bf16rne(v) * wab_v[sl]
                plsc.store_scatter(tbuf_v,
                                   [lanes, jnp.full((16,), j, jnp.int32)], acc)
            uvec = tbuf_v[0, :]
            for i in range(1, 16):
                uvec = uvec + tbuf_v[i, :]
            ex = jnp.exp(uvec * 2.0)
            a_v[pl.ds(gbase, 16)] = 1.0 - 2.0 / (ex + 1.0)
            return c2

        lax.fori_loop(0, EB // 16, group, 0)
        pltpu.sync_copy(a_v, a_out.at[pl.ds(e0, EB)])
        pltpu.sync_copy(a_v, asum_sp.at[srcv.at[0]], add=True)
        return carry

    lax.fori_loop(0, NBLK, block, 0)
    plsc.subcore_barrier()

    @pl.when(sid < NS - 1)
    def _():
        st = sid * 624
        pltpu.sync_copy(asum_sp.at[pl.ds(st, 624)], tmp_v.at[pl.ds(0, 624)])
        pltpu.sync_copy(tmp_v.at[pl.ds(0, 624)],
                        asum_out.at[pl.ds(cid * N + st, 624)])

    @pl.when(sid == NS - 1)
    def _():
        st = (NS - 1) * 624
        pltpu.sync_copy(asum_sp.at[pl.ds(st, 640)], tmp_v)
        pltpu.sync_copy(tmp_v, asum_out.at[pl.ds(cid * N + st, 640)])


def _sc2(tq, k, src, wab):
    mesh = plsc.VectorSubcoreMesh(core_axis_name="c", subcore_axis_name="s")
    fn = functools.partial(
        pl.kernel,
        out_type=(
            jax.ShapeDtypeStruct((E,), jnp.float32),
            jax.ShapeDtypeStruct((NC * N,), jnp.float32),
        ),
        mesh=mesh,
        compiler_params=pltpu.CompilerParams(needs_layout_passes=False),
        scratch_types=[
            pltpu.VMEM((1, EB), jnp.int32),
            pltpu.VMEM((EB, D), jnp.float32),
            pltpu.VMEM((EB, D), jnp.float32),
            pltpu.VMEM((EB,), jnp.float32),
            pltpu.VMEM((D,), jnp.float32),
            pltpu.VMEM((640,), jnp.float32),
            pltpu.VMEM((16, 16), jnp.float32),
            pltpu.VMEM_SHARED((N,), jnp.float32),
        ],
    )(_sc2_body)
    return fn(tq, k, src, wab)


def _sc3_body(f_hbm, a_hbm, src_hbm, asum_hbm,
              aggr_out,
              srcv, f_v, a_v, asum_v, tmp_v, zz_v,
              aggr_sp):
    cid = lax.axis_index("c")
    sid = lax.axis_index("s")
    tile_base = (cid * NS + sid) * EPT

    def zrow(r, c2):
        for c in range(D // 16):
            zz_v[r, pl.ds(c * 16, 16)] = jnp.zeros((16,), jnp.float32)
        return c2

    lax.fori_loop(0, 128, zrow, 0)
    for j in range(5):
        pltpu.sync_copy(zz_v, aggr_sp.at[pl.ds(sid * 640 + j * 128, 128)])

    pltpu.sync_copy(asum_hbm.at[pl.ds(0, N)], asum_v)
    pltpu.sync_copy(asum_hbm.at[pl.ds(N, N)], tmp_v)

    def arow(i, c2):
        sl = pl.ds(i * 16, 16)
        asum_v[sl] = asum_v[sl] + tmp_v[sl]
        return c2

    lax.fori_loop(0, N // 16, arow, 0)

    plsc.subcore_barrier()

    def block(blk, carry):
        e0 = tile_base + blk * EB
        pltpu.sync_copy(src_hbm.at[pl.ds(e0, EB)], srcv.at[0])
        pltpu.sync_copy(a_hbm.at[pl.ds(e0, EB)], a_v)
        pltpu.sync_copy(f_hbm.at[pl.ds(e0, EB)], f_v)

        def group(g, c2):
            gbase = g * 16
            sl0 = pl.ds(gbase, 16)
            idx16 = srcv[0, sl0]
            g16 = plsc.load_gather(asum_v, [idx16])
            s16 = jnp.exp(a_v[sl0] - g16)
            for j in range(16):
                e = gbase + j
                se = s16[j]
                for c in range(8):
                    sl = pl.ds(c * 16, 16)
                    f_v[e, sl] = f_v[e, sl] * se
            return c2

        lax.fori_loop(0, EB // 16, group, 0)

        pltpu.sync_copy(f_v, aggr_sp.at[srcv.at[0]], add=True)
        return carry

    lax.fori_loop(0, NBLK, block, 0)

    plsc.subcore_barrier()

    for j in range(5):
        r0 = sid * 640 + j * 128
        pltpu.sync_copy(aggr_sp.at[pl.ds(r0, 128)], zz_v)
        pltpu.sync_copy(zz_v, aggr_out.at[cid, pl.ds(r0, 128)])


def _sc3(f, a, src, asum):
    mesh = plsc.VectorSubcoreMesh(core_axis_name="c", subcore_axis_name="s")
    fn = functools.partial(
        pl.kernel,
        out_type=jax.ShapeDtypeStruct((NC, NP, D), jnp.float32),
        mesh=mesh,
        compiler_params=pltpu.CompilerParams(needs_layout_passes=False),
        scratch_types=[
            pltpu.VMEM((1, EB), jnp.int32),
            pltpu.VMEM((EB, D), jnp.float32),
            pltpu.VMEM((EB,), jnp.float32),
            pltpu.VMEM((N,), jnp.float32),
            pltpu.VMEM((N,), jnp.float32),
            pltpu.VMEM((128, D), jnp.float32),
            pltpu.VMEM_SHARED((NP, D), jnp.float32),
        ],
    )(_sc3_body)
    return fn(f, a, src, asum)


def kernel(x, edge_index, edge_attr, Wf, bf, Wq, Wk, Wa, Wu, bu):
    src = edge_index[0].astype(jnp.int32)
    dst = edge_index[1].astype(jnp.int32)

    w1 = Wf[0:D]
    w2 = Wf[D:2 * D]
    w3 = Wf[2 * D:3 * D]
    w4 = Wf[3 * D:]
    w4_pad = jnp.zeros((EFP, D), jnp.float32).at[:EF].set(w4)
    ea_pad = jnp.pad(edge_attr, ((0, 0), (0, EFP - EF)))
    bf_row = bf[None, :]
    bu_row = bu[None, :]
    wab = Wa[:, 0].astype(jnp.bfloat16).astype(jnp.float32)

    t1, t2, tq = _node_prep(x, w1, w2, w3, Wq)
    c, mask = _edge_prep(ea_pad, w4_pad, bf_row)
    f = _sc1(t1, t2, c, mask, src, dst)
    k = _kmm(f, Wk)
    a, asum = _sc2(tq, k, src, wab)
    aggr = _sc3(f, a, src, asum)
    return _final(x, aggr[:, :N], Wu, bu_row)


# speedup vs baseline: 1.8898x; 1.1350x over previous
import functools

import jax
import jax.numpy as jnp
from jax import lax
from jax.experimental import pallas as pl
from jax.experimental.pallas import tpu as pltpu
from jax.experimental.pallas import tpu_sc as plsc

N = 10000
E = 320000
D = 128
EF = 65
EFP = 128

NC = 2
NS = 16
NW = NC * NS
EPT = E // NW
EB = 80
NBLK = EPT // EB
NP = 10240

QK = float(D) ** (-0.5)


def _node_prep_body(x_ref, w1_ref, w2_ref, w3_ref, wq_ref,
                    t1_ref, t2_ref, tq_ref):
    xb = x_ref[...]
    a1 = jnp.dot(xb, w1_ref[...], preferred_element_type=jnp.float32)
    a2 = jnp.dot(xb, w2_ref[...], preferred_element_type=jnp.float32)
    a3 = jnp.dot(xb, w3_ref[...], preferred_element_type=jnp.float32)
    t1_ref[...] = a1 + a3
    t2_ref[...] = a2 - a3
    tq_ref[...] = jnp.dot(xb, wq_ref[...], preferred_element_type=jnp.float32)


def _node_prep(x, w1, w2, w3, wq):
    nb = 400
    grid = N // nb
    full = lambda shape: pl.BlockSpec(shape, lambda i: (0,) * len(shape))
    return pl.pallas_call(
        _node_prep_body,
        grid=(grid,),
        in_specs=[
            pl.BlockSpec((nb, D), lambda i: (i, 0)),
            full((D, D)), full((D, D)), full((D, D)), full((D, D)),
        ],
        out_specs=[
            pl.BlockSpec((nb, D), lambda i: (i, 0)),
            pl.BlockSpec((nb, D), lambda i: (i, 0)),
            pl.BlockSpec((nb, D), lambda i: (i, 0)),
        ],
        out_shape=[
            jax.ShapeDtypeStruct((N, D), jnp.float32),
            jax.ShapeDtypeStruct((N, D), jnp.float32),
            jax.ShapeDtypeStruct((N, D), jnp.float32),
        ],
    )(x, w1, w2, w3, wq)


def _edge_prep_body(ea_ref, w4_ref, bf_ref, c_ref, m_ref):
    eb = ea_ref[...]
    c_ref[...] = jnp.dot(eb, w4_ref[...],
                         preferred_element_type=jnp.float32) + bf_ref[...]
    m_ref[...] = (eb[:, 0] < 8.0).astype(jnp.float32)


def _edge_prep(ea_pad, w4_pad, bf):
    ebk = 512
    grid = E // ebk
    return pl.pallas_call(
        _edge_prep_body,
        grid=(grid,),
        in_specs=[
            pl.BlockSpec((ebk, EFP), lambda i: (i, 0)),
            pl.BlockSpec((EFP, D), lambda i: (0, 0)),
            pl.BlockSpec((1, D), lambda i: (0, 0)),
        ],
        out_specs=[
            pl.BlockSpec((ebk, D), lambda i: (i, 0)),
            pl.BlockSpec((ebk,), lambda i: (i,)),
        ],
        out_shape=[
            jax.ShapeDtypeStruct((E, D), jnp.float32),
            jax.ShapeDtypeStruct((E,), jnp.float32),
        ],
    )(ea_pad, w4_pad, bf)


def _amm_body(f_ref, qs_ref, wk_ref, wab_ref, a_ref):
    k = jnp.dot(f_ref[...], wk_ref[...], preferred_element_type=jnp.float32)
    v = (qs_ref[...] * k) * jnp.float32(QK)
    vb = v.astype(jnp.bfloat16)
    a_pre = jnp.dot(vb, wab_ref[...], preferred_element_type=jnp.float32)
    a_ref[...] = jnp.tanh(a_pre[:, 0])


def _amm(f, qs, wk, wab):
    ebk = 512
    grid = E // ebk
    return pl.pallas_call(
        _amm_body,
        grid=(grid,),
        in_specs=[
            pl.BlockSpec((ebk, D), lambda i: (i, 0)),
            pl.BlockSpec((ebk, D), lambda i: (i, 0)),
            pl.BlockSpec((D, D), lambda i: (0, 0)),
            pl.BlockSpec((D, 1), lambda i: (0, 0)),
        ],
        out_specs=pl.BlockSpec((ebk,), lambda i: (i,)),
        out_shape=jax.ShapeDtypeStruct((E,), jnp.float32),
    )(f, qs, wk, wab)


def _final_body(x_ref, ag_ref, wu_ref, bu_ref, o_ref):
    xa = x_ref[...] + ag_ref[0] + ag_ref[1]
    o_ref[...] = jnp.dot(xa, wu_ref[...],
                         preferred_element_type=jnp.float32) + bu_ref[...]


def _final(x, aggr, wu, bu):
    nb = 400
    grid = N // nb
    return pl.pallas_call(
        _final_body,
        grid=(grid,),
        in_specs=[
            pl.BlockSpec((nb, D), lambda i: (i, 0)),
            pl.BlockSpec((NC, nb, D), lambda i: (0, i, 0)),
            pl.BlockSpec((D, D), lambda i: (0, 0)),
            pl.BlockSpec((1, D), lambda i: (0, 0)),
        ],
        out_specs=pl.BlockSpec((nb, D), lambda i: (i, 0)),
        out_shape=jax.ShapeDtypeStruct((N, D), jnp.float32),
    )(x, aggr, wu, bu)


def _sc1_body(t1_hbm, t2_hbm, tq_hbm, c_hbm, mask_hbm, src_hbm, dst_hbm,
              f_out, qs_out,
              srcv, dstv, s_v, d_v, c_v, mask_v, f_v, q_v):
    cid = lax.axis_index("c")
    sid = lax.axis_index("s")
    tile_base = (cid * NS + sid) * EPT

    def block(blk, carry):
        e0 = tile_base + blk * EB
        pltpu.sync_copy(src_hbm.at[pl.ds(e0, EB)], srcv.at[0])
        pltpu.sync_copy(dst_hbm.at[pl.ds(e0, EB)], dstv.at[0])
        pltpu.sync_copy(t1_hbm.at[srcv.at[0]], s_v)
        pltpu.sync_copy(t2_hbm.at[dstv.at[0]], d_v)
        pltpu.sync_copy(tq_hbm.at[srcv.at[0]], q_v)
        pltpu.sync_copy(c_hbm.at[pl.ds(e0, EB)], c_v)
        pltpu.sync_copy(mask_hbm.at[pl.ds(e0, EB)], mask_v)

        def group(g, c2):
            gbase = g * 16
            m16 = mask_v[pl.ds(gbase, 16)]
            for j in range(16):
                e = gbase + j
                me = m16[j]
                for c in range(8):
                    sl = pl.ds(c * 16, 16)
                    pre = s_v[e, sl] + d_v[e, sl] + c_v[e, sl]
                    f_v[e, sl] = jnp.where(pre > 0.0, pre,
                                           jnp.exp(pre) - 1.0) * me
            return c2

        lax.fori_loop(0, EB // 16, group, 0)
        pltpu.sync_copy(f_v, f_out.at[pl.ds(e0, EB)])
        pltpu.sync_copy(q_v, qs_out.at[pl.ds(e0, EB)])
        return carry

    lax.fori_loop(0, NBLK, block, 0)


def _sc1(t1, t2, tq, c, mask, src, dst):
    mesh = plsc.VectorSubcoreMesh(core_axis_name="c", subcore_axis_name="s")
    fn = functools.partial(
        pl.kernel,
        out_type=(
            jax.ShapeDtypeStruct((E, D), jnp.float32),
            jax.ShapeDtypeStruct((E, D), jnp.float32),
        ),
        mesh=mesh,
        compiler_params=pltpu.CompilerParams(needs_layout_passes=False),
        scratch_types=[
            pltpu.VMEM((1, EB), jnp.int32),
            pltpu.VMEM((1, EB), jnp.int32),
            pltpu.VMEM((EB, D), jnp.float32),
            pltpu.VMEM((EB, D), jnp.float32),
            pltpu.VMEM((EB, D), jnp.float32),
            pltpu.VMEM((EB,), jnp.float32),
            pltpu.VMEM((EB, D), jnp.float32),
            pltpu.VMEM((EB, D), jnp.float32),
        ],
    )(_sc1_body)
    return fn(t1, t2, tq, c, mask, src, dst)


SB = 125
ROWS_PT = EPT // SB


def _sc2_body(a2_hbm, src2_hbm, asum_out,
              srcv, a_v, tmp_v, sem, asum_sp):
    cid = lax.axis_index("c")
    sid = lax.axis_index("s")
    r0 = (cid * NS + sid) * ROWS_PT

    for i in range(40):
        tmp_v[pl.ds(i * 16, 16)] = jnp.zeros((16,), jnp.float32)
    pltpu.sync_copy(tmp_v, asum_sp.at[pl.ds(sid * 624, 640)])
    pltpu.sync_copy(src2_hbm.at[pl.ds(r0, ROWS_PT)], srcv)
    pltpu.sync_copy(a2_hbm.at[pl.ds(r0, ROWS_PT)], a_v)
    plsc.subcore_barrier()

    descs = [
        pltpu.async_copy(a_v.at[j], asum_sp.at[srcv.at[j]], sem, add=True)
        for j in range(ROWS_PT)
    ]
    for d in descs:
        d.wait()

    plsc.subcore_barrier()

    @pl.when(sid < NS - 1)
    def _():
        st = sid * 624
        pltpu.sync_copy(asum_sp.at[pl.ds(st, 624)], tmp_v.at[pl.ds(0, 624)])
        pltpu.sync_copy(tmp_v.at[pl.ds(0, 624)],
                        asum_out.at[pl.ds(cid * N + st, 624)])

    @pl.when(sid == NS - 1)
    def _():
        st = (NS - 1) * 624
        pltpu.sync_copy(asum_sp.at[pl.ds(st, 640)], tmp_v)
        pltpu.sync_copy(tmp_v, asum_out.at[pl.ds(cid * N + st, 640)])


def _sc2(a2, src2):
    mesh = plsc.VectorSubcoreMesh(core_axis_name="c", subcore_axis_name="s")
    fn = functools.partial(
        pl.kernel,
        out_type=jax.ShapeDtypeStruct((NC * N,), jnp.float32),
        mesh=mesh,
        compiler_params=pltpu.CompilerParams(needs_layout_passes=False),
        scratch_types=[
            pltpu.VMEM((ROWS_PT, SB), jnp.int32),
            pltpu.VMEM((ROWS_PT, SB), jnp.float32),
            pltpu.VMEM((640,), jnp.float32),
            pltpu.SemaphoreType.DMA,
            pltpu.VMEM_SHARED((N,), jnp.float32),
        ],
    )(_sc2_body)
    return fn(a2, src2)


def _sc3_body(f_hbm, a_hbm, src_hbm, asum_hbm,
              aggr_out,
              srcv, f_v, a_v, asum_v, tmp_v, zz_v,
              aggr_sp):
    cid = lax.axis_index("c")
    sid = lax.axis_index("s")
    tile_base = (cid * NS + sid) * EPT

    def zrow(r, c2):
        for c in range(D // 16):
            zz_v[r, pl.ds(c * 16, 16)] = jnp.zeros((16,), jnp.float32)
        return c2

    lax.fori_loop(0, 128, zrow, 0)
    for j in range(5):
        pltpu.sync_copy(zz_v, aggr_sp.at[pl.ds(sid * 640 + j * 128, 128)])

    pltpu.sync_copy(asum_hbm.at[pl.ds(0, N)], asum_v)
    pltpu.sync_copy(asum_hbm.at[pl.ds(N, N)], tmp_v)

    def arow(i, c2):
        sl = pl.ds(i * 16, 16)
        asum_v[sl] = asum_v[sl] + tmp_v[sl]
        return c2

    lax.fori_loop(0, N // 16, arow, 0)

    plsc.subcore_barrier()

    def block(blk, carry):
        e0 = tile_base + blk * EB
        pltpu.sync_copy(src_hbm.at[pl.ds(e0, EB)], srcv.at[0])
        pltpu.sync_copy(a_hbm.at[pl.ds(e0, EB)], a_v)
        pltpu.sync_copy(f_hbm.at[pl.ds(e0, EB)], f_v)

        def group(g, c2):
            gbase = g * 16
            sl0 = pl.ds(gbase, 16)
            idx16 = srcv[0, sl0]
            g16 = plsc.load_gather(asum_v, [idx16])
            s16 = jnp.exp(a_v[sl0] - g16)
            for j in range(16):
                e = gbase + j
                se = s16[j]
                for c in range(8):
                    sl = pl.ds(c * 16, 16)
                    f_v[e, sl] = f_v[e, sl] * se
            return c2

        lax.fori_loop(0, EB // 16, group, 0)

        pltpu.sync_copy(f_v, aggr_sp.at[srcv.at[0]], add=True)
        return carry

    lax.fori_loop(0, NBLK, block, 0)

    plsc.subcore_barrier()

    for j in range(5):
        r0 = sid * 640 + j * 128
        pltpu.sync_copy(aggr_sp.at[pl.ds(r0, 128)], zz_v)
        pltpu.sync_copy(zz_v, aggr_out.at[cid, pl.ds(r0, 128)])


def _sc3(f, a, src, asum):
    mesh = plsc.VectorSubcoreMesh(core_axis_name="c", subcore_axis_name="s")
    fn = functools.partial(
        pl.kernel,
        out_type=jax.ShapeDtypeStruct((NC, NP, D), jnp.float32),
        mesh=mesh,
        compiler_params=pltpu.CompilerParams(needs_layout_passes=False),
        scratch_types=[
            pltpu.VMEM((1, EB), jnp.int32),
            pltpu.VMEM((EB, D), jnp.float32),
            pltpu.VMEM((EB,), jnp.float32),
            pltpu.VMEM((N,), jnp.float32),
            pltpu.VMEM((N,), jnp.float32),
            pltpu.VMEM((128, D), jnp.float32),
            pltpu.VMEM_SHARED((NP, D), jnp.float32),
        ],
    )(_sc3_body)
    return fn(f, a, src, asum)


def kernel(x, edge_index, edge_attr, Wf, bf, Wq, Wk, Wa, Wu, bu):
    src = edge_index[0].astype(jnp.int32)
    dst = edge_index[1].astype(jnp.int32)

    w1 = Wf[0:D]
    w2 = Wf[D:2 * D]
    w3 = Wf[2 * D:3 * D]
    w4 = Wf[3 * D:]
    w4_pad = jnp.zeros((EFP, D), jnp.float32).at[:EF].set(w4)
    ea_pad = jnp.pad(edge_attr, ((0, 0), (0, EFP - EF)))
    bf_row = bf[None, :]
    bu_row = bu[None, :]
    wab = Wa.astype(jnp.bfloat16)

    t1, t2, tq = _node_prep(x, w1, w2, w3, Wq)
    c, mask = _edge_prep(ea_pad, w4_pad, bf_row)
    f, qs = _sc1(t1, t2, tq, c, mask, src, dst)
    a = _amm(f, qs, Wk, wab)
    asum = _sc2(a.reshape(E // SB, SB), src.reshape(E // SB, SB))
    aggr = _sc3(f, a, src, asum)
    return _final(x, aggr, Wu, bu_row)


# speedup vs baseline: 1.9278x; 1.0201x over previous
import functools

import jax
import jax.numpy as jnp
from jax import lax
from jax.experimental import pallas as pl
from jax.experimental.pallas import tpu as pltpu
from jax.experimental.pallas import tpu_sc as plsc

N = 10000
E = 320000
D = 128
EF = 65
EFP = 128

NC = 2
NS = 16
NW = NC * NS
EPT = E // NW
EB = 80
NBLK = EPT // EB
NP = 10240

QK = float(D) ** (-0.5)


def _node_prep_body(x_ref, w1_ref, w2_ref, w3_ref, wq_ref,
                    t1_ref, t2_ref, tq_ref):
    xb = x_ref[...]
    a1 = jnp.dot(xb, w1_ref[...], preferred_element_type=jnp.float32)
    a2 = jnp.dot(xb, w2_ref[...], preferred_element_type=jnp.float32)
    a3 = jnp.dot(xb, w3_ref[...], preferred_element_type=jnp.float32)
    t1_ref[...] = a1 + a3
    t2_ref[...] = a2 - a3
    tq_ref[...] = jnp.dot(xb, wq_ref[...], preferred_element_type=jnp.float32)


def _node_prep(x, w1, w2, w3, wq):
    nb = 400
    grid = N // nb
    full = lambda shape: pl.BlockSpec(shape, lambda i: (0,) * len(shape))
    return pl.pallas_call(
        _node_prep_body,
        grid=(grid,),
        in_specs=[
            pl.BlockSpec((nb, D), lambda i: (i, 0)),
            full((D, D)), full((D, D)), full((D, D)), full((D, D)),
        ],
        out_specs=[
            pl.BlockSpec((nb, D), lambda i: (i, 0)),
            pl.BlockSpec((nb, D), lambda i: (i, 0)),
            pl.BlockSpec((nb, D), lambda i: (i, 0)),
        ],
        out_shape=[
            jax.ShapeDtypeStruct((N, D), jnp.float32),
            jax.ShapeDtypeStruct((N, D), jnp.float32),
            jax.ShapeDtypeStruct((N, D), jnp.float32),
        ],
    )(x, w1, w2, w3, wq)


def _edge_prep_body(ea_ref, w4_ref, bf_ref, c_ref, m_ref):
    eb = ea_ref[...]
    c_ref[...] = jnp.dot(eb, w4_ref[...],
                         preferred_element_type=jnp.float32) + bf_ref[...]
    m_ref[...] = (eb[:, 0] < 8.0).astype(jnp.float32)


def _edge_prep(ea, w4, bf):
    ebk = 512
    grid = E // ebk
    return pl.pallas_call(
        _edge_prep_body,
        grid=(grid,),
        in_specs=[
            pl.BlockSpec((ebk, EF), lambda i: (i, 0)),
            pl.BlockSpec((EF, D), lambda i: (0, 0)),
            pl.BlockSpec((1, D), lambda i: (0, 0)),
        ],
        out_specs=[
            pl.BlockSpec((ebk, D), lambda i: (i, 0)),
            pl.BlockSpec((ebk,), lambda i: (i,)),
        ],
        out_shape=[
            jax.ShapeDtypeStruct((E, D), jnp.float32),
            jax.ShapeDtypeStruct((E,), jnp.float32),
        ],
    )(ea, w4, bf)


def _amm_body(f_ref, qs_ref, wk_ref, wab_ref, a_ref):
    k = jnp.dot(f_ref[...], wk_ref[...], preferred_element_type=jnp.float32)
    v = (qs_ref[...] * k) * jnp.float32(QK)
    vb = v.astype(jnp.bfloat16)
    a_pre = jnp.dot(vb, wab_ref[...], preferred_element_type=jnp.float32)
    a_ref[...] = jnp.tanh(a_pre[:, 0])


def _amm(f, qs, wk, wab):
    ebk = 512
    grid = E // ebk
    return pl.pallas_call(
        _amm_body,
        grid=(grid,),
        in_specs=[
            pl.BlockSpec((ebk, D), lambda i: (i, 0)),
            pl.BlockSpec((ebk, D), lambda i: (i, 0)),
            pl.BlockSpec((D, D), lambda i: (0, 0)),
            pl.BlockSpec((D, 1), lambda i: (0, 0)),
        ],
        out_specs=pl.BlockSpec((ebk,), lambda i: (i,)),
        out_shape=jax.ShapeDtypeStruct((E,), jnp.float32),
    )(f, qs, wk, wab)


def _final_body(x_ref, ag_ref, wu_ref, bu_ref, o_ref):
    xa = x_ref[...] + ag_ref[0] + ag_ref[1]
    o_ref[...] = jnp.dot(xa, wu_ref[...],
                         preferred_element_type=jnp.float32) + bu_ref[...]


def _final(x, aggr, wu, bu):
    nb = 400
    grid = N // nb
    return pl.pallas_call(
        _final_body,
        grid=(grid,),
        in_specs=[
            pl.BlockSpec((nb, D), lambda i: (i, 0)),
            pl.BlockSpec((NC, nb, D), lambda i: (0, i, 0)),
            pl.BlockSpec((D, D), lambda i: (0, 0)),
            pl.BlockSpec((1, D), lambda i: (0, 0)),
        ],
        out_specs=pl.BlockSpec((nb, D), lambda i: (i, 0)),
        out_shape=jax.ShapeDtypeStruct((N, D), jnp.float32),
    )(x, aggr, wu, bu)


def _sc1_body(t1_hbm, t2_hbm, tq_hbm, c_hbm, mask_hbm, src_hbm, dst_hbm,
              f_out, qs_out,
              srcv, dstv, s_v, d_v, c_v, mask_v, f_v, q_v):
    cid = lax.axis_index("c")
    sid = lax.axis_index("s")
    tile_base = (cid * NS + sid) * EPT

    def block(blk, carry):
        e0 = tile_base + blk * EB
        pltpu.sync_copy(src_hbm.at[pl.ds(e0, EB)], srcv.at[0])
        pltpu.sync_copy(dst_hbm.at[pl.ds(e0, EB)], dstv.at[0])
        pltpu.sync_copy(t1_hbm.at[srcv.at[0]], s_v)
        pltpu.sync_copy(t2_hbm.at[dstv.at[0]], d_v)
        pltpu.sync_copy(tq_hbm.at[srcv.at[0]], q_v)
        pltpu.sync_copy(c_hbm.at[pl.ds(e0, EB)], c_v)
        pltpu.sync_copy(mask_hbm.at[pl.ds(e0, EB)], mask_v)

        def group(g, c2):
            gbase = g * 16
            m16 = mask_v[pl.ds(gbase, 16)]
            for j in range(16):
                e = gbase + j
                me = m16[j]
                for c in range(8):
                    sl = pl.ds(c * 16, 16)
                    pre = s_v[e, sl] + d_v[e, sl] + c_v[e, sl]
                    f_v[e, sl] = jnp.where(pre > 0.0, pre,
                                           jnp.exp(pre) - 1.0) * me
            return c2

        lax.fori_loop(0, EB // 16, group, 0)
        pltpu.sync_copy(f_v, f_out.at[pl.ds(e0, EB)])
        pltpu.sync_copy(q_v, qs_out.at[pl.ds(e0, EB)])
        return carry

    lax.fori_loop(0, NBLK, block, 0)


def _sc1(t1, t2, tq, c, mask, src, dst):
    mesh = plsc.VectorSubcoreMesh(core_axis_name="c", subcore_axis_name="s")
    fn = functools.partial(
        pl.kernel,
        out_type=(
            jax.ShapeDtypeStruct((E, D), jnp.float32),
            jax.ShapeDtypeStruct((E, D), jnp.float32),
        ),
        mesh=mesh,
        compiler_params=pltpu.CompilerParams(needs_layout_passes=False),
        scratch_types=[
            pltpu.VMEM((1, EB), jnp.int32),
            pltpu.VMEM((1, EB), jnp.int32),
            pltpu.VMEM((EB, D), jnp.float32),
            pltpu.VMEM((EB, D), jnp.float32),
            pltpu.VMEM((EB, D), jnp.float32),
            pltpu.VMEM((EB,), jnp.float32),
            pltpu.VMEM((EB, D), jnp.float32),
            pltpu.VMEM((EB, D), jnp.float32),
        ],
    )(_sc1_body)
    return fn(t1, t2, tq, c, mask, src, dst)


SB = 125
ROWS_PT = EPT // SB


def _sc2_body(a2_hbm, src2_hbm, asum_out,
              srcv, a_v, tmp_v, sem, asum_sp):
    cid = lax.axis_index("c")
    sid = lax.axis_index("s")
    r0 = (cid * NS + sid) * ROWS_PT

    for i in range(40):
        tmp_v[pl.ds(i * 16, 16)] = jnp.zeros((16,), jnp.float32)
    pltpu.sync_copy(tmp_v, asum_sp.at[pl.ds(sid * 624, 640)])
    pltpu.sync_copy(src2_hbm.at[pl.ds(r0, ROWS_PT)], srcv)
    pltpu.sync_copy(a2_hbm.at[pl.ds(r0, ROWS_PT)], a_v)
    plsc.subcore_barrier()

    descs = [
        pltpu.async_copy(a_v.at[j], asum_sp.at[srcv.at[j]], sem, add=True)
        for j in range(ROWS_PT)
    ]
    for d in descs:
        d.wait()

    plsc.subcore_barrier()

    @pl.when(sid < NS - 1)
    def _():
        st = sid * 624
        pltpu.sync_copy(asum_sp.at[pl.ds(st, 624)], tmp_v.at[pl.ds(0, 624)])
        pltpu.sync_copy(tmp_v.at[pl.ds(0, 624)],
                        asum_out.at[pl.ds(cid * N + st, 624)])

    @pl.when(sid == NS - 1)
    def _():
        st = (NS - 1) * 624
        pltpu.sync_copy(asum_sp.at[pl.ds(st, 640)], tmp_v)
        pltpu.sync_copy(tmp_v, asum_out.at[pl.ds(cid * N + st, 640)])


def _sc2(a2, src2):
    mesh = plsc.VectorSubcoreMesh(core_axis_name="c", subcore_axis_name="s")
    fn = functools.partial(
        pl.kernel,
        out_type=jax.ShapeDtypeStruct((NC * N,), jnp.float32),
        mesh=mesh,
        compiler_params=pltpu.CompilerParams(needs_layout_passes=False),
        scratch_types=[
            pltpu.VMEM((ROWS_PT, SB), jnp.int32),
            pltpu.VMEM((ROWS_PT, SB), jnp.float32),
            pltpu.VMEM((640,), jnp.float32),
            pltpu.SemaphoreType.DMA,
            pltpu.VMEM_SHARED((N,), jnp.float32),
        ],
    )(_sc2_body)
    return fn(a2, src2)


def _sc3_body(f_hbm, a_hbm, src_hbm, asum_hbm,
              aggr_out,
              srcv, f_v, a_v, asum_v, tmp_v, zz_v,
              aggr_sp):
    cid = lax.axis_index("c")
    sid = lax.axis_index("s")
    tile_base = (cid * NS + sid) * EPT

    def zrow(r, c2):
        for c in range(D // 16):
            zz_v[r, pl.ds(c * 16, 16)] = jnp.zeros((16,), jnp.float32)
        return c2

    lax.fori_loop(0, 128, zrow, 0)
    for j in range(5):
        pltpu.sync_copy(zz_v, aggr_sp.at[pl.ds(sid * 640 + j * 128, 128)])

    pltpu.sync_copy(asum_hbm.at[pl.ds(0, N)], asum_v)
    pltpu.sync_copy(asum_hbm.at[pl.ds(N, N)], tmp_v)

    def arow(i, c2):
        sl = pl.ds(i * 16, 16)
        asum_v[sl] = asum_v[sl] + tmp_v[sl]
        return c2

    lax.fori_loop(0, N // 16, arow, 0)

    plsc.subcore_barrier()

    def block(blk, carry):
        e0 = tile_base + blk * EB
        pltpu.sync_copy(src_hbm.at[pl.ds(e0, EB)], srcv.at[0])
        pltpu.sync_copy(a_hbm.at[pl.ds(e0, EB)], a_v)
        pltpu.sync_copy(f_hbm.at[pl.ds(e0, EB)], f_v)

        def group(g, c2):
            gbase = g * 16
            sl0 = pl.ds(gbase, 16)
            idx16 = srcv[0, sl0]
            g16 = plsc.load_gather(asum_v, [idx16])
            s16 = jnp.exp(a_v[sl0] - g16)
            for j in range(16):
                e = gbase + j
                se = s16[j]
                for c in range(8):
                    sl = pl.ds(c * 16, 16)
                    f_v[e, sl] = f_v[e, sl] * se
            return c2

        lax.fori_loop(0, EB // 16, group, 0)

        pltpu.sync_copy(f_v, aggr_sp.at[srcv.at[0]], add=True)
        return carry

    lax.fori_loop(0, NBLK, block, 0)

    plsc.subcore_barrier()

    for j in range(5):
        r0 = sid * 640 + j * 128
        pltpu.sync_copy(aggr_sp.at[pl.ds(r0, 128)], zz_v)
        pltpu.sync_copy(zz_v, aggr_out.at[cid, pl.ds(r0, 128)])


def _sc3(f, a, src, asum):
    mesh = plsc.VectorSubcoreMesh(core_axis_name="c", subcore_axis_name="s")
    fn = functools.partial(
        pl.kernel,
        out_type=jax.ShapeDtypeStruct((NC, NP, D), jnp.float32),
        mesh=mesh,
        compiler_params=pltpu.CompilerParams(needs_layout_passes=False),
        scratch_types=[
            pltpu.VMEM((1, EB), jnp.int32),
            pltpu.VMEM((EB, D), jnp.float32),
            pltpu.VMEM((EB,), jnp.float32),
            pltpu.VMEM((N,), jnp.float32),
            pltpu.VMEM((N,), jnp.float32),
            pltpu.VMEM((128, D), jnp.float32),
            pltpu.VMEM_SHARED((NP, D), jnp.float32),
        ],
    )(_sc3_body)
    return fn(f, a, src, asum)


def kernel(x, edge_index, edge_attr, Wf, bf, Wq, Wk, Wa, Wu, bu):
    src = edge_index[0].astype(jnp.int32)
    dst = edge_index[1].astype(jnp.int32)

    w1 = Wf[0:D]
    w2 = Wf[D:2 * D]
    w3 = Wf[2 * D:3 * D]
    w4 = Wf[3 * D:]
    bf_row = bf[None, :]
    bu_row = bu[None, :]
    wab = Wa.astype(jnp.bfloat16)

    t1, t2, tq = _node_prep(x, w1, w2, w3, Wq)
    c, mask = _edge_prep(edge_attr, w4, bf_row)
    f, qs = _sc1(t1, t2, tq, c, mask, src, dst)
    a = _amm(f, qs, Wk, wab)
    asum = _sc2(a.reshape(E // SB, SB), src.reshape(E // SB, SB))
    aggr = _sc3(f, a, src, asum)
    return _final(x, aggr, Wu, bu_row)


# speedup vs baseline: 2.4258x; 1.2583x over previous
import functools

import jax
import jax.numpy as jnp
from jax import lax
from jax.experimental import pallas as pl
from jax.experimental.pallas import tpu as pltpu
from jax.experimental.pallas import tpu_sc as plsc

N = 10000
E = 320000
D = 128
EF = 65
EFP = 128

NC = 2
NS = 16
NW = NC * NS
EPT = E // NW
EB = 80
NBLK = EPT // EB
NP = 10240

QK = float(D) ** (-0.5)


def _node_prep_body(x_ref, w1_ref, w2_ref, w3_ref, wq_ref,
                    t1_ref, t2_ref, tq_ref):
    xb = x_ref[...]
    a1 = jnp.dot(xb, w1_ref[...], preferred_element_type=jnp.float32)
    a2 = jnp.dot(xb, w2_ref[...], preferred_element_type=jnp.float32)
    a3 = jnp.dot(xb, w3_ref[...], preferred_element_type=jnp.float32)
    t1_ref[...] = a1 + a3
    t2_ref[...] = a2 - a3
    tq_ref[...] = jnp.dot(xb, wq_ref[...], preferred_element_type=jnp.float32)


def _node_prep(x, w1, w2, w3, wq):
    nb = 400
    grid = N // nb
    full = lambda shape: pl.BlockSpec(shape, lambda i: (0,) * len(shape))
    return pl.pallas_call(
        _node_prep_body,
        grid=(grid,),
        in_specs=[
            pl.BlockSpec((nb, D), lambda i: (i, 0)),
            full((D, D)), full((D, D)), full((D, D)), full((D, D)),
        ],
        out_specs=[
            pl.BlockSpec((nb, D), lambda i: (i, 0)),
            pl.BlockSpec((nb, D), lambda i: (i, 0)),
            pl.BlockSpec((nb, D), lambda i: (i, 0)),
        ],
        out_shape=[
            jax.ShapeDtypeStruct((N, D), jnp.float32),
            jax.ShapeDtypeStruct((N, D), jnp.float32),
            jax.ShapeDtypeStruct((N, D), jnp.float32),
        ],
    )(x, w1, w2, w3, wq)


def _edge_prep_body(ea_ref, w4_ref, bf_ref, c_ref, m_ref):
    eb = ea_ref[...]
    c_ref[...] = jnp.dot(eb, w4_ref[...],
                         preferred_element_type=jnp.float32) + bf_ref[...]
    m_ref[...] = (eb[:, 0] < 8.0).astype(jnp.float32)


def _edge_prep(ea, w4, bf):
    ebk = 512
    grid = E // ebk
    return pl.pallas_call(
        _edge_prep_body,
        grid=(grid,),
        in_specs=[
            pl.BlockSpec((ebk, EF), lambda i: (i, 0)),
            pl.BlockSpec((EF, D), lambda i: (0, 0)),
            pl.BlockSpec((1, D), lambda i: (0, 0)),
        ],
        out_specs=[
            pl.BlockSpec((ebk, D), lambda i: (i, 0)),
            pl.BlockSpec((ebk,), lambda i: (i,)),
        ],
        out_shape=[
            jax.ShapeDtypeStruct((E, D), jnp.float32),
            jax.ShapeDtypeStruct((E,), jnp.float32),
        ],
    )(ea, w4, bf)


def _amm_body(f_ref, qs_ref, wk_ref, wab_ref, a_ref):
    k = jnp.dot(f_ref[...], wk_ref[...], preferred_element_type=jnp.float32)
    v = (qs_ref[...] * k) * jnp.float32(QK)
    vb = v.astype(jnp.bfloat16)
    a_pre = jnp.dot(vb, wab_ref[...], preferred_element_type=jnp.float32)
    a_ref[...] = jnp.tanh(a_pre[:, 0])


def _amm(f, qs, wk, wab):
    ebk = 512
    grid = E // ebk
    return pl.pallas_call(
        _amm_body,
        grid=(grid,),
        in_specs=[
            pl.BlockSpec((ebk, D), lambda i: (i, 0)),
            pl.BlockSpec((ebk, D), lambda i: (i, 0)),
            pl.BlockSpec((D, D), lambda i: (0, 0)),
            pl.BlockSpec((D, 1), lambda i: (0, 0)),
        ],
        out_specs=pl.BlockSpec((ebk,), lambda i: (i,)),
        out_shape=jax.ShapeDtypeStruct((E,), jnp.float32),
    )(f, qs, wk, wab)


def _final_body(x_ref, ag_ref, wu_ref, bu_ref, o_ref):
    xa = x_ref[...] + ag_ref[0] + ag_ref[1]
    o_ref[...] = jnp.dot(xa, wu_ref[...],
                         preferred_element_type=jnp.float32) + bu_ref[...]


def _final(x, aggr, wu, bu):
    nb = 400
    grid = N // nb
    return pl.pallas_call(
        _final_body,
        grid=(grid,),
        in_specs=[
            pl.BlockSpec((nb, D), lambda i: (i, 0)),
            pl.BlockSpec((NC, nb, D), lambda i: (0, i, 0)),
            pl.BlockSpec((D, D), lambda i: (0, 0)),
            pl.BlockSpec((1, D), lambda i: (0, 0)),
        ],
        out_specs=pl.BlockSpec((nb, D), lambda i: (i, 0)),
        out_shape=jax.ShapeDtypeStruct((N, D), jnp.float32),
    )(x, aggr, wu, bu)


def _sc1_body(t1_hbm, t2_hbm, tq_hbm, c_hbm, mask_hbm, src_hbm, dst_hbm,
              f_out, qs_out,
              srcv, dstv, s_v, d_v, c_v, mask_v, f_v, q_v, sem):
    cid = lax.axis_index("c")
    sid = lax.axis_index("s")
    tile_base = (cid * NS + sid) * EPT

    def block(blk, carry):
        e0 = tile_base + blk * EB
        di = pltpu.async_copy(src_hbm.at[pl.ds(e0, EB)], srcv.at[0], sem)
        dj = pltpu.async_copy(dst_hbm.at[pl.ds(e0, EB)], dstv.at[0], sem)
        d4 = pltpu.async_copy(c_hbm.at[pl.ds(e0, EB)], c_v, sem)
        d5 = pltpu.async_copy(mask_hbm.at[pl.ds(e0, EB)], mask_v, sem)
        di.wait()
        dj.wait()
        d1 = pltpu.async_copy(t1_hbm.at[srcv.at[0]], s_v, sem)
        d2 = pltpu.async_copy(t2_hbm.at[dstv.at[0]], d_v, sem)
        d3 = pltpu.async_copy(tq_hbm.at[srcv.at[0]], q_v, sem)
        d4.wait()
        d5.wait()
        d1.wait()
        d2.wait()
        d3.wait()

        def group(g, c2):
            gbase = g * 16
            m16 = mask_v[pl.ds(gbase, 16)]
            for j in range(16):
                e = gbase + j
                me = m16[j]
                for c in range(8):
                    sl = pl.ds(c * 16, 16)
                    pre = s_v[e, sl] + d_v[e, sl] + c_v[e, sl]
                    f_v[e, sl] = jnp.where(pre > 0.0, pre,
                                           jnp.exp(pre) - 1.0) * me
            return c2

        lax.fori_loop(0, EB // 16, group, 0)
        d6 = pltpu.async_copy(f_v, f_out.at[pl.ds(e0, EB)], sem)
        d7 = pltpu.async_copy(q_v, qs_out.at[pl.ds(e0, EB)], sem)
        d6.wait()
        d7.wait()
        return carry

    lax.fori_loop(0, NBLK, block, 0)


def _sc1(t1, t2, tq, c, mask, src, dst):
    mesh = plsc.VectorSubcoreMesh(core_axis_name="c", subcore_axis_name="s")
    fn = functools.partial(
        pl.kernel,
        out_type=(
            jax.ShapeDtypeStruct((E, D), jnp.float32),
            jax.ShapeDtypeStruct((E, D), jnp.float32),
        ),
        mesh=mesh,
        compiler_params=pltpu.CompilerParams(needs_layout_passes=False),
        scratch_types=[
            pltpu.VMEM((1, EB), jnp.int32),
            pltpu.VMEM((1, EB), jnp.int32),
            pltpu.VMEM((EB, D), jnp.float32),
            pltpu.VMEM((EB, D), jnp.float32),
            pltpu.VMEM((EB, D), jnp.float32),
            pltpu.VMEM((EB,), jnp.float32),
            pltpu.VMEM((EB, D), jnp.float32),
            pltpu.VMEM((EB, D), jnp.float32),
            pltpu.SemaphoreType.DMA,
        ],
    )(_sc1_body)
    return fn(t1, t2, tq, c, mask, src, dst)


SB = 125
ROWS_PT = EPT // SB


def _sc2_body(a2_hbm, src2_hbm, asum_out,
              srcv, a_v, tmp_v, sem, asum_sp):
    cid = lax.axis_index("c")
    sid = lax.axis_index("s")
    r0 = (cid * NS + sid) * ROWS_PT

    for i in range(40):
        tmp_v[pl.ds(i * 16, 16)] = jnp.zeros((16,), jnp.float32)
    pltpu.sync_copy(tmp_v, asum_sp.at[pl.ds(sid * 624, 640)])
    pltpu.sync_copy(src2_hbm.at[pl.ds(r0, ROWS_PT)], srcv)
    pltpu.sync_copy(a2_hbm.at[pl.ds(r0, ROWS_PT)], a_v)
    plsc.subcore_barrier()

    descs = [
        pltpu.async_copy(a_v.at[j], asum_sp.at[srcv.at[j]], sem, add=True)
        for j in range(ROWS_PT)
    ]
    for d in descs:
        d.wait()

    plsc.subcore_barrier()

    @pl.when(sid < NS - 1)
    def _():
        st = sid * 624
        pltpu.sync_copy(asum_sp.at[pl.ds(st, 624)], tmp_v.at[pl.ds(0, 624)])
        pltpu.sync_copy(tmp_v.at[pl.ds(0, 624)],
                        asum_out.at[pl.ds(cid * N + st, 624)])

    @pl.when(sid == NS - 1)
    def _():
        st = (NS - 1) * 624
        pltpu.sync_copy(asum_sp.at[pl.ds(st, 640)], tmp_v)
        pltpu.sync_copy(tmp_v, asum_out.at[pl.ds(cid * N + st, 640)])


def _sc2(a2, src2):
    mesh = plsc.VectorSubcoreMesh(core_axis_name="c", subcore_axis_name="s")
    fn = functools.partial(
        pl.kernel,
        out_type=jax.ShapeDtypeStruct((NC * N,), jnp.float32),
        mesh=mesh,
        compiler_params=pltpu.CompilerParams(needs_layout_passes=False),
        scratch_types=[
            pltpu.VMEM((ROWS_PT, SB), jnp.int32),
            pltpu.VMEM((ROWS_PT, SB), jnp.float32),
            pltpu.VMEM((640,), jnp.float32),
            pltpu.SemaphoreType.DMA,
            pltpu.VMEM_SHARED((N,), jnp.float32),
        ],
    )(_sc2_body)
    return fn(a2, src2)


def _sc3_body(f_hbm, a_hbm, src_hbm, asum_hbm,
              aggr_out,
              srcv, f_v, a_v, asum_v, tmp_v, zz_v, sem,
              aggr_sp):
    cid = lax.axis_index("c")
    sid = lax.axis_index("s")
    tile_base = (cid * NS + sid) * EPT

    def zrow(r, c2):
        for c in range(D // 16):
            zz_v[r, pl.ds(c * 16, 16)] = jnp.zeros((16,), jnp.float32)
        return c2

    lax.fori_loop(0, 128, zrow, 0)
    for j in range(5):
        pltpu.sync_copy(zz_v, aggr_sp.at[pl.ds(sid * 640 + j * 128, 128)])

    pltpu.sync_copy(asum_hbm.at[pl.ds(0, N)], asum_v)
    pltpu.sync_copy(asum_hbm.at[pl.ds(N, N)], tmp_v)

    def arow(i, c2):
        sl = pl.ds(i * 16, 16)
        asum_v[sl] = asum_v[sl] + tmp_v[sl]
        return c2

    lax.fori_loop(0, N // 16, arow, 0)

    plsc.subcore_barrier()

    def block(blk, carry):
        e0 = tile_base + blk * EB
        d1 = pltpu.async_copy(src_hbm.at[pl.ds(e0, EB)], srcv.at[0], sem)
        d2 = pltpu.async_copy(a_hbm.at[pl.ds(e0, EB)], a_v, sem)
        d3 = pltpu.async_copy(f_hbm.at[pl.ds(e0, EB)], f_v, sem)
        d1.wait()
        d2.wait()
        d3.wait()

        def group(g, c2):
            gbase = g * 16
            sl0 = pl.ds(gbase, 16)
            idx16 = srcv[0, sl0]
            g16 = plsc.load_gather(asum_v, [idx16])
            s16 = jnp.exp(a_v[sl0] - g16)
            for j in range(16):
                e = gbase + j
                se = s16[j]
                for c in range(8):
                    sl = pl.ds(c * 16, 16)
                    f_v[e, sl] = f_v[e, sl] * se
            return c2

        lax.fori_loop(0, EB // 16, group, 0)

        pltpu.sync_copy(f_v, aggr_sp.at[srcv.at[0]], add=True)
        return carry

    lax.fori_loop(0, NBLK, block, 0)

    plsc.subcore_barrier()

    for j in range(5):
        r0 = sid * 640 + j * 128
        pltpu.sync_copy(aggr_sp.at[pl.ds(r0, 128)], zz_v)
        pltpu.sync_copy(zz_v, aggr_out.at[cid, pl.ds(r0, 128)])


def _sc3(f, a, src, asum):
    mesh = plsc.VectorSubcoreMesh(core_axis_name="c", subcore_axis_name="s")
    fn = functools.partial(
        pl.kernel,
        out_type=jax.ShapeDtypeStruct((NC, NP, D), jnp.float32),
        mesh=mesh,
        compiler_params=pltpu.CompilerParams(needs_layout_passes=False),
        scratch_types=[
            pltpu.VMEM((1, EB), jnp.int32),
            pltpu.VMEM((EB, D), jnp.float32),
            pltpu.VMEM((EB,), jnp.float32),
            pltpu.VMEM((N,), jnp.float32),
            pltpu.VMEM((N,), jnp.float32),
            pltpu.VMEM((128, D), jnp.float32),
            pltpu.SemaphoreType.DMA,
            pltpu.VMEM_SHARED((NP, D), jnp.float32),
        ],
    )(_sc3_body)
    return fn(f, a, src, asum)


def kernel(x, edge_index, edge_attr, Wf, bf, Wq, Wk, Wa, Wu, bu):
    src = edge_index[0].astype(jnp.int32)
    dst = edge_index[1].astype(jnp.int32)

    w1 = Wf[0:D]
    w2 = Wf[D:2 * D]
    w3 = Wf[2 * D:3 * D]
    w4 = Wf[3 * D:]
    bf_row = bf[None, :]
    bu_row = bu[None, :]
    wab = Wa.astype(jnp.bfloat16)

    t1, t2, tq = _node_prep(x, w1, w2, w3, Wq)
    c, mask = _edge_prep(edge_attr, w4, bf_row)
    f, qs = _sc1(t1, t2, tq, c, mask, src, dst)
    a = _amm(f, qs, Wk, wab)
    asum = _sc2(a.reshape(E // SB, SB), src.reshape(E // SB, SB))
    aggr = _sc3(f, a, src, asum)
    return _final(x, aggr, Wu, bu_row)
